# baseline (speedup 1.0000x reference)
import functools

import jax
import jax.numpy as jnp
from jax import lax
from jax.experimental import pallas as pl
from jax.experimental.pallas import tpu as pltpu
from jax.experimental.pallas import tpu_sc as plsc

_LANE = 128
_R = 8


@functools.cache
def _build(nrows: int, vocab: int, dim: int):
    info = plsc.get_sparse_core_info()
    nw = info.num_cores * info.num_subcores
    assert nrows % nw == 0
    rows_per_w = nrows // nw
    assert rows_per_w % _R == 0
    nchunks = rows_per_w // _R

    mesh = plsc.VectorSubcoreMesh(core_axis_name="c", subcore_axis_name="s")

    @functools.partial(
        pl.kernel,
        mesh=mesh,
        out_type=jax.ShapeDtypeStruct((nrows, _LANE, dim), jnp.float32),
        scratch_types=[
            pltpu.VMEM((_R, _LANE), jnp.int32),
            pltpu.VMEM((_R, _LANE, dim), jnp.float32),
            pltpu.SemaphoreType.DMA,
        ],
        compiler_params=pltpu.CompilerParams(use_tc_tiling_on_sc=False),
    )
    def gather_kernel(ids_hbm, table_hbm, out_hbm, idx_v, rows_v, sem):
        wid = lax.axis_index("s") * info.num_cores + lax.axis_index("c")
        base = wid * rows_per_w

        def chunk_body(c, carry):
            row0 = base + c * _R
            pltpu.sync_copy(ids_hbm.at[pl.ds(row0, _R)], idx_v)
            cps = [
                pltpu.async_copy(table_hbm.at[idx_v.at[j]], rows_v.at[j], sem)
                for j in range(_R)
            ]
            for cp in cps:
                cp.wait()
            pltpu.sync_copy(rows_v, out_hbm.at[pl.ds(row0, _R)])
            return carry

        lax.fori_loop(0, nchunks, chunk_body, 0)

    return gather_kernel


def kernel(input_ids, table):
    batch, hist = input_ids.shape
    vocab, dim = table.shape
    total = batch * hist
    nrows = total // _LANE
    ids = input_ids.reshape(nrows, _LANE).astype(jnp.int32)
    out = _build(nrows, vocab, dim)(ids, table)
    return out.reshape(batch, hist, dim)


# speedup vs baseline: 1.0315x; 1.0315x over previous
import functools

import jax
import jax.numpy as jnp
from jax import lax
from jax.experimental import pallas as pl
from jax.experimental.pallas import tpu as pltpu
from jax.experimental.pallas import tpu_sc as plsc

_LANE = 128
_R = 5
_NBUF = 2


@functools.cache
def _build(nrows: int, vocab: int, dim: int):
    info = plsc.get_sparse_core_info()
    nw = info.num_cores * info.num_subcores
    assert nrows % nw == 0
    rows_per_w = nrows // nw
    assert rows_per_w % (_R * _NBUF) == 0
    nchunks = rows_per_w // _R

    mesh = plsc.VectorSubcoreMesh(core_axis_name="c", subcore_axis_name="s")

    @functools.partial(
        pl.kernel,
        mesh=mesh,
        out_type=jax.ShapeDtypeStruct((nrows, _LANE, dim), jnp.float32),
        scratch_types=[
            [pltpu.VMEM((_R, _LANE), jnp.int32)] * _NBUF,
            [pltpu.VMEM((_R, _LANE, dim), jnp.float32)] * _NBUF,
            [pltpu.SemaphoreType.DMA] * _NBUF,
            [pltpu.SemaphoreType.DMA] * _NBUF,
            [pltpu.SemaphoreType.DMA] * _NBUF,
        ],
        compiler_params=pltpu.CompilerParams(use_tc_tiling_on_sc=False),
    )
    def gather_kernel(ids_hbm, table_hbm, out_hbm, idx_v, rows_v, isem, gsem, osem):
        wid = lax.axis_index("s") * info.num_cores + lax.axis_index("c")
        base = wid * rows_per_w

        def idx_copy(c, b):
            return pltpu.make_async_copy(
                ids_hbm.at[pl.ds(base + c * _R, _R)], idx_v[b], isem[b])

        def out_copy(c, b):
            return pltpu.make_async_copy(
                rows_v[b], out_hbm.at[pl.ds(base + c * _R, _R)], osem[b])

        idx_copy(0, 0).start()

        def pair_body(cc, carry):
            for b in range(_NBUF):
                c = cc * _NBUF + b
                @pl.when(c >= _NBUF)
                def _():
                    out_copy(c, b).wait()
                idx_copy(c, b).wait()
                @pl.when(c + 1 < nchunks)
                def _():
                    idx_copy(c + 1, (b + 1) % _NBUF).start()
                cps = [
                    pltpu.async_copy(
                        table_hbm.at[idx_v[b].at[j]], rows_v[b].at[j], gsem[b])
                    for j in range(_R)
                ]
                for cp in cps:
                    cp.wait()
                out_copy(c, b).start()
            return carry

        lax.fori_loop(0, nchunks // _NBUF, pair_body, 0)
        for b in range(_NBUF):
            out_copy(nchunks - _NBUF + b, b).wait()

    return gather_kernel


def kernel(input_ids, table):
    batch, hist = input_ids.shape
    vocab, dim = table.shape
    total = batch * hist
    nrows = total // _LANE
    ids = input_ids.reshape(nrows, _LANE).astype(jnp.int32)
    out = _build(nrows, vocab, dim)(ids, table)
    return out.reshape(batch, hist, dim)
